# speedup vs baseline: 3.0479x; 3.0479x over previous
import jax
import jax.numpy as jnp
from jax.experimental import pallas as pl
from jax.experimental.pallas import tpu as pltpu

N_BASES = 4
L = 1024
BI = 128


def _body(tab_ref, seqc_ref, seqr_ref, out_ref):
    seqc = seqc_ref[...]
    seqr = seqr_ref[...]
    for c in range(N_BASES):
        acc_i = jnp.zeros((BI, 1), jnp.float32)
        acc_j = jnp.zeros((1, L), jnp.float32)
        for k in range(N_BASES):
            t = tab_ref[k, c]
            acc_i += t * (seqc == k).astype(jnp.float32)
            acc_j += t * (seqr == k).astype(jnp.float32)
        out_ref[c] = jnp.broadcast_to(acc_i, (BI, L))
        out_ref[N_BASES + c] = jnp.broadcast_to(acc_j, (BI, L))


def kernel(sequence, base_table):
    seq_col = sequence.reshape(L, 1)
    seq_row = sequence.reshape(1, L)
    out = pl.pallas_call(
        _body,
        grid=(L // BI,),
        in_specs=[
            pl.BlockSpec(memory_space=pltpu.SMEM),
            pl.BlockSpec((BI, 1), lambda i: (i, 0)),
            pl.BlockSpec((1, L), lambda i: (0, 0)),
        ],
        out_specs=pl.BlockSpec((2 * N_BASES, BI, L), lambda i: (0, i, 0)),
        out_shape=jax.ShapeDtypeStruct((2 * N_BASES, L, L), jnp.float32),
    )(base_table, seq_col, seq_row)
    return out[None]
